# baseline (speedup 1.0000x reference)
import functools

import jax
import jax.numpy as jnp
from jax import lax
from jax.experimental import pallas as pl
from jax.experimental.pallas import tpu as pltpu
from jax.experimental.pallas import tpu_sc as plsc

N_NODES = 10000
NPAD = 10240
NDAG = 100
NWRK = 50
CH = 128
NCH = 80
NTILES = 32
EPAD = NTILES * NCH * CH
DUMP = N_NODES
BLK = 1024
GRID = NPAD // BLK
RPT = NPAD // 16


@functools.cache
def _sc_kernels():
    mesh = plsc.VectorSubcoreMesh(core_axis_name="c", subcore_axis_name="s")

    @functools.partial(
        pl.kernel,
        out_type=jax.ShapeDtypeStruct((2, NPAD), jnp.float32),
        mesh=mesh,
        scratch_types=[
            pltpu.VMEM((NCH, CH), jnp.int32),
            pltpu.VMEM((CH,), jnp.float32),
            pltpu.VMEM_SHARED((NPAD,), jnp.float32),
            pltpu.SemaphoreType.DMA,
        ],
    )
    def sc_degree(col_hbm, zeros_hbm, ones_hbm, out_hbm, idx_v, ones_v,
                  deg_sh, sem):
        c = lax.axis_index("c")
        s = lax.axis_index("s")
        wid = c * 16 + s
        pltpu.sync_copy(col_hbm.at[pl.ds(wid * NCH, NCH)], idx_v)
        pltpu.sync_copy(ones_hbm, ones_v)
        pltpu.sync_copy(zeros_hbm, deg_sh.at[pl.ds(s * RPT, RPT)])
        plsc.subcore_barrier()

        def fire(j, carry):
            pltpu.async_copy(ones_v, deg_sh.at[idx_v.at[j]], sem, add=True)
            return carry

        lax.fori_loop(0, NCH, fire, 0)

        def drain(j, carry):
            pltpu.make_async_copy(ones_v, deg_sh.at[idx_v.at[j]], sem).wait()
            return carry

        lax.fori_loop(0, NCH, drain, 0)
        plsc.subcore_barrier()
        pltpu.sync_copy(deg_sh.at[pl.ds(s * RPT, RPT)],
                        out_hbm.at[c, pl.ds(s * RPT, RPT)])

    @functools.partial(
        pl.kernel,
        out_type=jax.ShapeDtypeStruct((2, NPAD, 8), jnp.float32),
        mesh=mesh,
        scratch_types=[
            pltpu.VMEM((NCH, CH), jnp.int32),
            pltpu.VMEM((NCH, CH), jnp.int32),
            pltpu.VMEM((NCH, CH, 8), jnp.float32),
            pltpu.VMEM_SHARED((NPAD, 8), jnp.float32),
            pltpu.SemaphoreType.DMA,
            pltpu.SemaphoreType.DMA,
        ],
        compiler_params=pltpu.CompilerParams(use_tc_tiling_on_sc=False),
    )
    def sc_gather_scatter(col_hbm, row_hbm, g_hbm, zeros_hbm, out_hbm,
                          col_v, row_v, rows_v, acc_sh, sem, sem2):
        c = lax.axis_index("c")
        s = lax.axis_index("s")
        wid = c * 16 + s
        pltpu.sync_copy(col_hbm.at[pl.ds(wid * NCH, NCH)], col_v)
        pltpu.sync_copy(row_hbm.at[pl.ds(wid * NCH, NCH)], row_v)
        pltpu.sync_copy(zeros_hbm, acc_sh.at[pl.ds(s * RPT, RPT)])

        def fire(j, carry):
            pltpu.async_copy(g_hbm.at[col_v.at[j]], rows_v.at[j], sem)
            return carry

        lax.fori_loop(0, NCH, fire, 0)

        def drain(j, carry):
            pltpu.make_async_copy(g_hbm.at[col_v.at[j]], rows_v.at[j],
                                  sem).wait()
            return carry

        lax.fori_loop(0, NCH, drain, 0)
        plsc.subcore_barrier()

        def scat(j, carry):
            pltpu.async_copy(rows_v.at[j], acc_sh.at[row_v.at[j]], sem2,
                             add=True)
            return carry

        lax.fori_loop(0, NCH, scat, 0)

        def sdrain(j, carry):
            pltpu.make_async_copy(rows_v.at[j], acc_sh.at[row_v.at[j]],
                                  sem2).wait()
            return carry

        lax.fori_loop(0, NCH, sdrain, 0)
        plsc.subcore_barrier()
        pltpu.sync_copy(acc_sh.at[pl.ds(s * RPT, RPT)],
                        out_hbm.at[c, pl.ds(s * RPT, RPT)])

    return sc_degree, sc_gather_scatter


def _sc_degree(col_r, zeros1, ones_c):
    return _sc_kernels()[0](col_r, zeros1, ones_c)


def _sc_gather_scatter(col_r, row_r, g, zeros8):
    return _sc_kernels()[1](col_r, row_r, g, zeros8)


def _dot(a, b):
    return jax.lax.dot_general(a, b, (((a.ndim - 1,), (0,)), ((), ())))


def _dot_hp(a, b):
    return jax.lax.dot_general(a, b, (((a.ndim - 1,), (0,)), ((), ())),
                               precision=jax.lax.Precision.HIGHEST)


def _b16(v):
    return v.astype(jnp.bfloat16).astype(jnp.float32)


def _k1_body(x_ref, b3_ref, dp_ref, wa, ba, wb, bb, wc, bc,
             g_ref, dis_ref, yx_ref):
    i = pl.program_id(0)
    xb = x_ref[...]
    a = jnp.maximum(_dot(xb, wa[...].T) + ba[...], 0.0)
    a = jnp.maximum(_dot(a, wb[...].T) + bb[...], 0.0)
    h = _dot(a, wc[...].T) + bc[...]
    deg = dp_ref[0] + dp_ref[1] + 1.0
    dis = 1.0 / jnp.sqrt(deg)
    dis_ref[...] = dis
    g_ref[...] = dis * h
    onehot = (lax.broadcasted_iota(jnp.int32, (NDAG, BLK), 0)
              == b3_ref[0]).astype(jnp.float32)

    @pl.when(i == 0)
    def _():
        yx_ref[...] = jnp.zeros_like(yx_ref)

    yx_ref[...] += _dot_hp(onehot, xb)


def _k3_body(sp_ref, g_ref, dis_ref, b3_ref, wa, ba, wb, bb, wc, bc,
             xn_ref, yn_ref):
    i = pl.program_id(0)
    aggr = dis_ref[...] * (sp_ref[0] + sp_ref[1] + g_ref[...])
    a = jnp.maximum(_dot(aggr, wa[...].T) + ba[...], 0.0)
    a = jnp.maximum(_dot(a, wb[...].T) + bb[...], 0.0)
    xn = _dot(a, wc[...].T) + bc[...]
    xn_ref[...] = xn
    onehot = (lax.broadcasted_iota(jnp.int32, (NDAG, BLK), 0)
              == b3_ref[0]).astype(jnp.float32)

    @pl.when(i == 0)
    def _():
        yn_ref[...] = jnp.zeros_like(yn_ref)

    yn_ref[...] += _dot_hp(onehot, xn)


def _k45_body(xn_ref, b3_ref, yx_ref, yn_ref,
              wd1x, wd1n, bd1, wd2, bd2, wd3, bd3,
              wg1, bg1, wg2, bg2, wg3, bg3,
              wpl, wpy, wpz, bp1, wp2, bp2, wp3, bp3,
              wax, way, waz, ba, wb, bb, wc, bc,
              ops_ref, pr_ref):
    i = pl.program_id(0)
    y1 = jnp.maximum(_dot(yx_ref[...], wd1x[...].T)
                     + _dot(yn_ref[...], wd1n[...].T) + bd1[...], 0.0)
    y2 = jnp.maximum(_dot(y1, wd2[...].T) + bd2[...], 0.0)
    y = _dot(y2, wd3[...].T) + bd3[...]
    z0 = jnp.sum(y, axis=0, keepdims=True)
    z1 = jnp.maximum(_dot(z0, wg1[...].T) + bg1[...], 0.0)
    z2 = jnp.maximum(_dot(z1, wg2[...].T) + bg2[...], 0.0)
    z = _dot(z2, wg3[...].T) + bg3[...]

    @pl.when(i == 0)
    def _():
        t_y = _dot(y, wpy[...].T)
        t_z = _dot(z, wpz[...].T) + bp1[...]
        lim = (lax.broadcasted_iota(jnp.int32, (NWRK, 32), 0) + 1
               ).astype(jnp.float32)
        t_l = lim * wpl[...]
        l1 = jnp.maximum(t_y[:, None, :] + t_l[None, :, :] + t_z[None], 0.0)
        l2 = jnp.maximum(
            lax.dot_general(l1, wp2[...], (((2,), (1,)), ((), ())))
            + bp2[...], 0.0)
        pr = jnp.sum(_b16(l2) * _b16(wp3[...])[None], axis=2)
        pr_ref[...] = pr + bp3[...]

    onehot = (lax.broadcasted_iota(jnp.int32, (NDAG, BLK), 0)
              == b3_ref[0]).astype(jnp.float32)
    yb = lax.dot_general(onehot, y, (((0,), (0,)), ((), ())),
                         precision=jax.lax.Precision.HIGHEST)
    l1o = jnp.maximum(
        _dot(xn_ref[...], wax[...].T) + _dot(yb, way[...].T)
        + _dot(z, waz[...].T) + ba[...], 0.0)
    l2o = jnp.maximum(_dot(l1o, wb[...].T) + bb[...], 0.0)
    ops_ref[...] = (jnp.sum(_b16(l2o) * _b16(wc[...]), axis=1, keepdims=True)
                    + bc[...])


def _full(shape):
    return pl.BlockSpec(shape, lambda i: tuple(0 for _ in shape))


def kernel(x, edge_index, batch, num_ops_per_dag, op_msk, prlvl_msk, params):
    f32 = jnp.float32
    (w1a, b1a), (w1b, b1b), (w1c, b1c) = params['mlp1']
    (w2a, b2a), (w2b, b2b), (w2c, b2c) = params['mlp2']
    (wda, bda), (wdb, bdb), (wdc, bdc) = params['mlp_dag']
    (wga, bga), (wgb, bgb), (wgc, bgc) = params['mlp_global']
    (woa, boa), (wob, bob), (woc, boc) = params['mlp_op_score']
    (wpa, bpa), (wpb, bpb), (wpc, bpc) = params['mlp_prlvl_score']
    r2 = lambda b: b.reshape(1, -1)

    x_p = jnp.pad(x, ((0, NPAD - N_NODES), (0, 0)))
    batch_p = jnp.pad(batch, (0, NPAD - N_NODES), constant_values=NDAG + 1)
    b3 = batch_p.reshape(GRID, 1, BLK)
    row = edge_index[0]
    col = edge_index[1]
    pad_n = EPAD - row.shape[0]
    row_r = jnp.pad(row, (0, pad_n), constant_values=DUMP).reshape(-1, CH)
    col_r = jnp.pad(col, (0, pad_n), constant_values=DUMP).reshape(-1, CH)
    zeros1 = jnp.zeros((RPT,), f32)
    zeros8 = jnp.zeros((RPT, 8), f32)
    ones_c = jnp.ones((CH,), f32)

    deg_parts = _sc_degree(col_r, zeros1, ones_c)
    dp = deg_parts.reshape(2, NPAD, 1)

    g, dis, y_x = pl.pallas_call(
        _k1_body,
        grid=(GRID,),
        in_specs=[
            pl.BlockSpec((BLK, 128), lambda i: (i, 0)),
            pl.BlockSpec((1, 1, BLK), lambda i: (i, 0, 0)),
            pl.BlockSpec((2, BLK, 1), lambda i: (0, i, 0)),
            _full((32, 128)), _full((1, 32)),
            _full((16, 32)), _full((1, 16)),
            _full((8, 16)), _full((1, 8)),
        ],
        out_specs=[
            pl.BlockSpec((BLK, 8), lambda i: (i, 0)),
            pl.BlockSpec((BLK, 1), lambda i: (i, 0)),
            _full((NDAG, 128)),
        ],
        out_shape=[
            jax.ShapeDtypeStruct((NPAD, 8), f32),
            jax.ShapeDtypeStruct((NPAD, 1), f32),
            jax.ShapeDtypeStruct((NDAG, 128), f32),
        ],
    )(x_p, b3, dp, w1a, r2(b1a), w1b, r2(b1b), w1c, r2(b1c))

    s_parts = _sc_gather_scatter(col_r, row_r, g, zeros8)

    x_node, y_n = pl.pallas_call(
        _k3_body,
        grid=(GRID,),
        in_specs=[
            pl.BlockSpec((2, BLK, 8), lambda i: (0, i, 0)),
            pl.BlockSpec((BLK, 8), lambda i: (i, 0)),
            pl.BlockSpec((BLK, 1), lambda i: (i, 0)),
            pl.BlockSpec((1, 1, BLK), lambda i: (i, 0, 0)),
            _full((32, 8)), _full((1, 32)),
            _full((16, 32)), _full((1, 16)),
            _full((16, 16)), _full((1, 16)),
        ],
        out_specs=[
            pl.BlockSpec((BLK, 16), lambda i: (i, 0)),
            _full((NDAG, 16)),
        ],
        out_shape=[
            jax.ShapeDtypeStruct((NPAD, 16), f32),
            jax.ShapeDtypeStruct((NDAG, 16), f32),
        ],
    )(s_parts, g, dis, b3, w2a, r2(b2a), w2b, r2(b2b), w2c, r2(b2c))

    ops, prlvl = pl.pallas_call(
        _k45_body,
        grid=(GRID,),
        in_specs=[
            pl.BlockSpec((BLK, 16), lambda i: (i, 0)),
            pl.BlockSpec((1, 1, BLK), lambda i: (i, 0, 0)),
            _full((NDAG, 128)), _full((NDAG, 16)),
            _full((32, 128)), _full((32, 16)), _full((1, 32)),
            _full((16, 32)), _full((1, 16)),
            _full((16, 16)), _full((1, 16)),
            _full((32, 16)), _full((1, 32)),
            _full((16, 32)), _full((1, 16)),
            _full((16, 16)), _full((1, 16)),
            _full((1, 32)), _full((32, 16)), _full((32, 16)), _full((1, 32)),
            _full((16, 32)), _full((1, 16)),
            _full((1, 16)), _full((1, 1)),
            _full((32, 16)), _full((32, 16)), _full((32, 16)), _full((1, 32)),
            _full((16, 32)), _full((1, 16)),
            _full((1, 16)), _full((1, 1)),
        ],
        out_specs=[
            pl.BlockSpec((BLK, 1), lambda i: (i, 0)),
            _full((NDAG, NWRK)),
        ],
        out_shape=[
            jax.ShapeDtypeStruct((NPAD, 1), f32),
            jax.ShapeDtypeStruct((NDAG, NWRK), f32),
        ],
    )(x_node, b3, y_x, y_n,
      wda[:, :128], wda[:, 128:], r2(bda),
      wdb, r2(bdb), wdc, r2(bdc),
      wga, r2(bga), wgb, r2(bgb), wgc, r2(bgc),
      wpa[:, 0:1].T, wpa[:, 1:17], wpa[:, 17:33], r2(bpa),
      wpb, r2(bpb), wpc, bpc.reshape(1, 1),
      woa[:, 0:16], woa[:, 16:32], woa[:, 32:48], r2(boa),
      wob, r2(bob), woc, boc.reshape(1, 1))

    return ops[:N_NODES, 0], prlvl


# speedup vs baseline: 1.2127x; 1.2127x over previous
import functools

import jax
import jax.numpy as jnp
from jax import lax
from jax.experimental import pallas as pl
from jax.experimental.pallas import tpu as pltpu
from jax.experimental.pallas import tpu_sc as plsc

N_NODES = 10000
NPAD = 10240
NDAG = 100
NWRK = 50
CH = 80
NCH = 125
NTILES = 32
EPT = NCH * CH
BLK = 1000
GRID = N_NODES // BLK
RPT = NPAD // 16
RPS = N_NODES // 16


@functools.cache
def _sc_kernels():
    mesh = plsc.VectorSubcoreMesh(core_axis_name="c", subcore_axis_name="s")

    @functools.partial(
        pl.kernel,
        out_type=jax.ShapeDtypeStruct((2, NPAD), jnp.float32),
        mesh=mesh,
        scratch_types=[
            pltpu.VMEM((EPT,), jnp.int32),
            pltpu.VMEM((EPT,), jnp.float32),
            pltpu.VMEM_SHARED((NPAD,), jnp.float32),
            pltpu.SemaphoreType.DMA,
        ],
    )
    def sc_degree(col_hbm, zeros_hbm, ones_hbm, out_hbm, idx_v, ones_v,
                  deg_sh, sem):
        c = lax.axis_index("c")
        s = lax.axis_index("s")
        wid = c * 16 + s
        pltpu.async_copy(col_hbm.at[pl.ds(wid * EPT, EPT)], idx_v, sem)
        pltpu.async_copy(ones_hbm, ones_v, sem)
        pltpu.sync_copy(zeros_hbm, deg_sh.at[pl.ds(s * RPT, RPT)])
        pltpu.make_async_copy(col_hbm.at[pl.ds(wid * EPT, EPT)], idx_v,
                              sem).wait()
        pltpu.make_async_copy(ones_hbm, ones_v, sem).wait()
        plsc.subcore_barrier()
        pltpu.sync_copy(ones_v, deg_sh.at[idx_v], add=True)
        plsc.subcore_barrier()
        pltpu.sync_copy(deg_sh.at[pl.ds(s * RPT, RPT)],
                        out_hbm.at[c, pl.ds(s * RPT, RPT)])

    @functools.partial(
        pl.kernel,
        out_type=jax.ShapeDtypeStruct((2, N_NODES, 8), jnp.float32),
        mesh=mesh,
        scratch_types=[
            pltpu.VMEM((EPT,), jnp.int32),
            pltpu.VMEM((EPT,), jnp.int32),
            pltpu.VMEM((EPT, 8), jnp.float32),
            pltpu.VMEM_SHARED((N_NODES, 8), jnp.float32),
            pltpu.SemaphoreType.DMA,
        ],
        compiler_params=pltpu.CompilerParams(use_tc_tiling_on_sc=False),
    )
    def sc_gather_scatter(col_hbm, row_hbm, g_hbm, zeros_hbm, out_hbm,
                          col_v, row_v, rows_v, acc_sh, sem):
        c = lax.axis_index("c")
        s = lax.axis_index("s")
        wid = c * 16 + s
        pltpu.async_copy(col_hbm.at[pl.ds(wid * EPT, EPT)], col_v, sem)
        pltpu.async_copy(row_hbm.at[pl.ds(wid * EPT, EPT)], row_v, sem)
        pltpu.sync_copy(zeros_hbm, acc_sh.at[pl.ds(s * RPS, RPS)])
        pltpu.make_async_copy(col_hbm.at[pl.ds(wid * EPT, EPT)], col_v,
                              sem).wait()
        pltpu.make_async_copy(row_hbm.at[pl.ds(wid * EPT, EPT)], row_v,
                              sem).wait()
        pltpu.async_copy(g_hbm.at[col_v], rows_v, sem).wait()
        plsc.subcore_barrier()
        pltpu.sync_copy(rows_v, acc_sh.at[row_v], add=True)
        plsc.subcore_barrier()
        pltpu.sync_copy(acc_sh.at[pl.ds(s * RPS, RPS)],
                        out_hbm.at[c, pl.ds(s * RPS, RPS)])

    return sc_degree, sc_gather_scatter


def _sc_degree(col1, zeros1, ones2):
    return _sc_kernels()[0](col1, zeros1, ones2)


def _sc_gather_scatter(col1, row1, g, zeros8):
    return _sc_kernels()[1](col1, row1, g, zeros8)


def _dot(a, b):
    return jax.lax.dot_general(a, b, (((a.ndim - 1,), (0,)), ((), ())))


def _dot_hp(a, b):
    return jax.lax.dot_general(a, b, (((a.ndim - 1,), (0,)), ((), ())),
                               precision=jax.lax.Precision.HIGHEST)


def _b16(v):
    return v.astype(jnp.bfloat16).astype(jnp.float32)


def _dot_oh(onehot, x):
    dims = (((1,), (0,)), ((), ()))
    oh = onehot.astype(jnp.bfloat16)
    x1 = x.astype(jnp.bfloat16)
    r1 = x - x1.astype(jnp.float32)
    x2 = r1.astype(jnp.bfloat16)
    x3 = (r1 - x2.astype(jnp.float32)).astype(jnp.bfloat16)
    f32 = jnp.float32
    return ((jax.lax.dot_general(oh, x3, dims, preferred_element_type=f32)
             + jax.lax.dot_general(oh, x2, dims, preferred_element_type=f32))
            + jax.lax.dot_general(oh, x1, dims, preferred_element_type=f32))


def _k1_body(x_ref, b3_ref, dp_ref, wa, ba, wb, bb, wc, bc,
             g_ref, dis_ref, yx_ref):
    i = pl.program_id(0)
    xb = x_ref[...]
    a = jnp.maximum(_dot(xb, wa[...].T) + ba[...], 0.0)
    a = jnp.maximum(_dot(a, wb[...].T) + bb[...], 0.0)
    h = _dot(a, wc[...].T) + bc[...]
    deg = dp_ref[0] + dp_ref[1] + 1.0
    dis = 1.0 / jnp.sqrt(deg)
    dis_ref[...] = dis
    g_ref[...] = dis * h
    onehot = (lax.broadcasted_iota(jnp.int32, (NDAG, BLK), 0)
              == b3_ref[0]).astype(jnp.float32)

    @pl.when(i == 0)
    def _():
        yx_ref[...] = jnp.zeros_like(yx_ref)

    yx_ref[...] += _dot_oh(onehot, xb)


def _k3_body(sp_ref, g_ref, dis_ref, b3_ref, wa, ba, wb, bb, wc, bc,
             xn_ref, yn_ref):
    i = pl.program_id(0)
    aggr = dis_ref[...] * (sp_ref[0] + sp_ref[1] + g_ref[...])
    a = jnp.maximum(_dot(aggr, wa[...].T) + ba[...], 0.0)
    a = jnp.maximum(_dot(a, wb[...].T) + bb[...], 0.0)
    xn = _dot(a, wc[...].T) + bc[...]
    xn_ref[...] = xn
    onehot = (lax.broadcasted_iota(jnp.int32, (NDAG, BLK), 0)
              == b3_ref[0]).astype(jnp.float32)

    @pl.when(i == 0)
    def _():
        yn_ref[...] = jnp.zeros_like(yn_ref)

    yn_ref[...] += _dot_hp(onehot, xn)


def _k45_body(xn_ref, b3_ref, yx_ref, yn_ref,
              wd1x, wd1n, bd1, wd2, bd2, wd3, bd3,
              wg1, bg1, wg2, bg2, wg3, bg3,
              wpl, wpy, wpz, bp1, wp2, bp2, wp3, bp3,
              wax, way, waz, ba, wb, bb, wc, bc,
              ops_ref, pr_ref):
    i = pl.program_id(0)
    y1 = jnp.maximum(_dot(yx_ref[...], wd1x[...].T)
                     + _dot(yn_ref[...], wd1n[...].T) + bd1[...], 0.0)
    y2 = jnp.maximum(_dot(y1, wd2[...].T) + bd2[...], 0.0)
    y = _dot(y2, wd3[...].T) + bd3[...]
    z0 = jnp.sum(y, axis=0, keepdims=True)
    z1 = jnp.maximum(_dot(z0, wg1[...].T) + bg1[...], 0.0)
    z2 = jnp.maximum(_dot(z1, wg2[...].T) + bg2[...], 0.0)
    z = _dot(z2, wg3[...].T) + bg3[...]

    @pl.when(i == 0)
    def _():
        t_y = _dot(y, wpy[...].T)
        t_z = _dot(z, wpz[...].T) + bp1[...]
        lim = (lax.broadcasted_iota(jnp.int32, (NWRK, 32), 0) + 1
               ).astype(jnp.float32)
        t_l = lim * wpl[...]
        l1 = jnp.maximum(t_y[:, None, :] + t_l[None, :, :] + t_z[None], 0.0)
        l2 = jnp.maximum(
            lax.dot_general(l1, wp2[...], (((2,), (1,)), ((), ())))
            + bp2[...], 0.0)
        pr = jnp.sum(_b16(l2) * _b16(wp3[...])[None], axis=2)
        pr_ref[...] = pr + bp3[...]

    onehot = (lax.broadcasted_iota(jnp.int32, (NDAG, BLK), 0)
              == b3_ref[0]).astype(jnp.float32)
    yb = lax.dot_general(onehot, y, (((0,), (0,)), ((), ())),
                         precision=jax.lax.Precision.HIGHEST)
    l1o = jnp.maximum(
        _dot(xn_ref[...], wax[...].T) + _dot(yb, way[...].T)
        + _dot(z, waz[...].T) + ba[...], 0.0)
    l2o = jnp.maximum(_dot(l1o, wb[...].T) + bb[...], 0.0)
    ops_ref[...] = (jnp.sum(_b16(l2o) * _b16(wc[...]), axis=1, keepdims=True)
                    + bc[...])


def _full(shape):
    return pl.BlockSpec(shape, lambda i: tuple(0 for _ in shape))


def kernel(x, edge_index, batch, num_ops_per_dag, op_msk, prlvl_msk, params):
    f32 = jnp.float32
    (w1a, b1a), (w1b, b1b), (w1c, b1c) = params['mlp1']
    (w2a, b2a), (w2b, b2b), (w2c, b2c) = params['mlp2']
    (wda, bda), (wdb, bdb), (wdc, bdc) = params['mlp_dag']
    (wga, bga), (wgb, bgb), (wgc, bgc) = params['mlp_global']
    (woa, boa), (wob, bob), (woc, boc) = params['mlp_op_score']
    (wpa, bpa), (wpb, bpb), (wpc, bpc) = params['mlp_prlvl_score']
    r2 = lambda b: b.reshape(1, -1)

    b3 = batch.reshape(GRID, 1, BLK)
    row1 = edge_index[0]
    col1 = edge_index[1]
    zeros1 = jnp.zeros((RPT,), f32)
    zeros8 = jnp.zeros((RPS, 8), f32)
    ones2 = jnp.ones((EPT,), f32)

    deg_parts = _sc_degree(col1, zeros1, ones2)
    dp = deg_parts.reshape(2, NPAD, 1)

    g, dis, y_x = pl.pallas_call(
        _k1_body,
        grid=(GRID,),
        in_specs=[
            pl.BlockSpec((BLK, 128), lambda i: (i, 0)),
            pl.BlockSpec((1, 1, BLK), lambda i: (i, 0, 0)),
            pl.BlockSpec((2, BLK, 1), lambda i: (0, i, 0)),
            _full((32, 128)), _full((1, 32)),
            _full((16, 32)), _full((1, 16)),
            _full((8, 16)), _full((1, 8)),
        ],
        out_specs=[
            pl.BlockSpec((BLK, 8), lambda i: (i, 0)),
            pl.BlockSpec((BLK, 1), lambda i: (i, 0)),
            _full((NDAG, 128)),
        ],
        out_shape=[
            jax.ShapeDtypeStruct((N_NODES, 8), f32),
            jax.ShapeDtypeStruct((N_NODES, 1), f32),
            jax.ShapeDtypeStruct((NDAG, 128), f32),
        ],
    )(x, b3, dp, w1a, r2(b1a), w1b, r2(b1b), w1c, r2(b1c))

    s_parts = _sc_gather_scatter(col1, row1, g, zeros8)

    x_node, y_n = pl.pallas_call(
        _k3_body,
        grid=(GRID,),
        in_specs=[
            pl.BlockSpec((2, BLK, 8), lambda i: (0, i, 0)),
            pl.BlockSpec((BLK, 8), lambda i: (i, 0)),
            pl.BlockSpec((BLK, 1), lambda i: (i, 0)),
            pl.BlockSpec((1, 1, BLK), lambda i: (i, 0, 0)),
            _full((32, 8)), _full((1, 32)),
            _full((16, 32)), _full((1, 16)),
            _full((16, 16)), _full((1, 16)),
        ],
        out_specs=[
            pl.BlockSpec((BLK, 16), lambda i: (i, 0)),
            _full((NDAG, 16)),
        ],
        out_shape=[
            jax.ShapeDtypeStruct((N_NODES, 16), f32),
            jax.ShapeDtypeStruct((NDAG, 16), f32),
        ],
    )(s_parts, g, dis, b3, w2a, r2(b2a), w2b, r2(b2b), w2c, r2(b2c))

    ops, prlvl = pl.pallas_call(
        _k45_body,
        grid=(GRID,),
        in_specs=[
            pl.BlockSpec((BLK, 16), lambda i: (i, 0)),
            pl.BlockSpec((1, 1, BLK), lambda i: (i, 0, 0)),
            _full((NDAG, 128)), _full((NDAG, 16)),
            _full((32, 128)), _full((32, 16)), _full((1, 32)),
            _full((16, 32)), _full((1, 16)),
            _full((16, 16)), _full((1, 16)),
            _full((32, 16)), _full((1, 32)),
            _full((16, 32)), _full((1, 16)),
            _full((16, 16)), _full((1, 16)),
            _full((1, 32)), _full((32, 16)), _full((32, 16)), _full((1, 32)),
            _full((16, 32)), _full((1, 16)),
            _full((1, 16)), _full((1, 1)),
            _full((32, 16)), _full((32, 16)), _full((32, 16)), _full((1, 32)),
            _full((16, 32)), _full((1, 16)),
            _full((1, 16)), _full((1, 1)),
        ],
        out_specs=[
            pl.BlockSpec((BLK, 1), lambda i: (i, 0)),
            _full((NDAG, NWRK)),
        ],
        out_shape=[
            jax.ShapeDtypeStruct((N_NODES, 1), f32),
            jax.ShapeDtypeStruct((NDAG, NWRK), f32),
        ],
    )(x_node, b3, y_x, y_n,
      wda[:, :128], wda[:, 128:], r2(bda),
      wdb, r2(bdb), wdc, r2(bdc),
      wga, r2(bga), wgb, r2(bgb), wgc, r2(bgc),
      wpa[:, 0:1].T, wpa[:, 1:17], wpa[:, 17:33], r2(bpa),
      wpb, r2(bpb), wpc, bpc.reshape(1, 1),
      woa[:, 0:16], woa[:, 16:32], woa[:, 32:48], r2(boa),
      wob, r2(bob), woc, boc.reshape(1, 1))

    return ops[:, 0], prlvl


# speedup vs baseline: 1.2209x; 1.0067x over previous
import functools

import jax
import jax.numpy as jnp
from jax import lax
from jax.experimental import pallas as pl
from jax.experimental.pallas import tpu as pltpu
from jax.experimental.pallas import tpu_sc as plsc

N_NODES = 10000
NPAD = 10240
NDAG = 100
NWRK = 50
CH = 80
NCH = 125
NTILES = 32
EPT = NCH * CH
BLK = 1000
GRID = N_NODES // BLK
RPT = NPAD // 16
RPS = N_NODES // 16


@functools.cache
def _sc_kernels():
    mesh = plsc.VectorSubcoreMesh(core_axis_name="c", subcore_axis_name="s")

    @functools.partial(
        pl.kernel,
        out_type=jax.ShapeDtypeStruct((2, NPAD), jnp.float32),
        mesh=mesh,
        scratch_types=[
            pltpu.VMEM((EPT,), jnp.int32),
            pltpu.VMEM((EPT,), jnp.float32),
            pltpu.VMEM_SHARED((NPAD,), jnp.float32),
            pltpu.SemaphoreType.DMA,
        ],
    )
    def sc_degree(col_hbm, zeros_hbm, ones_hbm, out_hbm, idx_v, ones_v,
                  deg_sh, sem):
        c = lax.axis_index("c")
        s = lax.axis_index("s")
        wid = c * 16 + s
        pltpu.async_copy(col_hbm.at[pl.ds(wid * EPT, EPT)], idx_v, sem)
        pltpu.async_copy(ones_hbm, ones_v, sem)
        pltpu.sync_copy(zeros_hbm, deg_sh.at[pl.ds(s * RPT, RPT)])
        pltpu.make_async_copy(col_hbm.at[pl.ds(wid * EPT, EPT)], idx_v,
                              sem).wait()
        pltpu.make_async_copy(ones_hbm, ones_v, sem).wait()
        plsc.subcore_barrier()
        pltpu.sync_copy(ones_v, deg_sh.at[idx_v], add=True)
        plsc.subcore_barrier()
        pltpu.sync_copy(deg_sh.at[pl.ds(s * RPT, RPT)],
                        out_hbm.at[c, pl.ds(s * RPT, RPT)])

    @functools.partial(
        pl.kernel,
        out_type=jax.ShapeDtypeStruct((2, N_NODES, 8), jnp.float32),
        mesh=mesh,
        scratch_types=[
            pltpu.VMEM((EPT,), jnp.int32),
            pltpu.VMEM((EPT,), jnp.int32),
            pltpu.VMEM((EPT, 8), jnp.float32),
            pltpu.VMEM_SHARED((N_NODES, 8), jnp.float32),
            pltpu.SemaphoreType.DMA,
        ],
        compiler_params=pltpu.CompilerParams(use_tc_tiling_on_sc=False),
    )
    def sc_gather_scatter(col_hbm, row_hbm, g_hbm, zeros_hbm, out_hbm,
                          col_v, row_v, rows_v, acc_sh, sem):
        c = lax.axis_index("c")
        s = lax.axis_index("s")
        wid = c * 16 + s
        pltpu.async_copy(col_hbm.at[pl.ds(wid * EPT, EPT)], col_v, sem)
        pltpu.async_copy(row_hbm.at[pl.ds(wid * EPT, EPT)], row_v, sem)
        pltpu.sync_copy(zeros_hbm, acc_sh.at[pl.ds(s * RPS, RPS)])
        pltpu.make_async_copy(col_hbm.at[pl.ds(wid * EPT, EPT)], col_v,
                              sem).wait()
        pltpu.make_async_copy(row_hbm.at[pl.ds(wid * EPT, EPT)], row_v,
                              sem).wait()
        pltpu.async_copy(g_hbm.at[col_v], rows_v, sem).wait()
        plsc.subcore_barrier()
        pltpu.sync_copy(rows_v, acc_sh.at[row_v], add=True)
        plsc.subcore_barrier()
        pltpu.sync_copy(acc_sh.at[pl.ds(s * RPS, RPS)],
                        out_hbm.at[c, pl.ds(s * RPS, RPS)])

    return sc_degree, sc_gather_scatter


def _sc_degree(col1, zeros1, ones2):
    return _sc_kernels()[0](col1, zeros1, ones2)


def _sc_gather_scatter(col1, row1, g, zeros8):
    return _sc_kernels()[1](col1, row1, g, zeros8)


def _dot(a, b):
    return jax.lax.dot_general(a, b, (((a.ndim - 1,), (0,)), ((), ())))


def _dot_hp(a, b):
    return jax.lax.dot_general(a, b, (((a.ndim - 1,), (0,)), ((), ())),
                               precision=jax.lax.Precision.HIGHEST)


def _b16(v):
    return v.astype(jnp.bfloat16).astype(jnp.float32)


def _dot_oh(onehot, x):
    dims = (((1,), (0,)), ((), ()))
    oh = onehot.astype(jnp.bfloat16)
    x1 = x.astype(jnp.bfloat16)
    r1 = x - x1.astype(jnp.float32)
    x2 = r1.astype(jnp.bfloat16)
    x3 = (r1 - x2.astype(jnp.float32)).astype(jnp.bfloat16)
    f32 = jnp.float32
    return ((jax.lax.dot_general(oh, x3, dims, preferred_element_type=f32)
             + jax.lax.dot_general(oh, x2, dims, preferred_element_type=f32))
            + jax.lax.dot_general(oh, x1, dims, preferred_element_type=f32))


def _k1_body(x_ref, b3_ref, dp_ref, wa, ba, wb, bb, wc, bc,
             g_ref, dis_ref, yx_ref):
    i = pl.program_id(0)
    xb = x_ref[...]
    a = jnp.maximum(_dot(xb, wa[...].T) + ba[...], 0.0)
    a = jnp.maximum(_dot(a, wb[...].T) + bb[...], 0.0)
    h = _dot(a, wc[...].T) + bc[...]
    deg = dp_ref[0] + dp_ref[1] + 1.0
    dis = 1.0 / jnp.sqrt(deg)
    dis_ref[...] = dis
    g_ref[...] = dis * h
    onehot = (lax.broadcasted_iota(jnp.int32, (NDAG, BLK), 0)
              == b3_ref[0]).astype(jnp.float32)

    @pl.when(i == 0)
    def _():
        yx_ref[...] = jnp.zeros_like(yx_ref)

    yx_ref[...] += _dot_oh(onehot, xb)


def _k345_body(sp_ref, g_ref, dis_ref, b3_ref, yx_ref,
               wa, ba, wb, bb, wc, bc,
               wd1x, wd1n, bd1, wd2, bd2, wd3, bd3,
               wg1, bg1, wg2, bg2, wg3, bg3,
               wpl, wpy, wpz, bp1, wp2, bp2, wp3, bp3,
               wax, way, waz, boa, wob, bob, woc, boc,
               ops_ref, pr_ref, xn_s, yn_s):
    p = pl.program_id(0)
    i = pl.program_id(1)
    onehot = (lax.broadcasted_iota(jnp.int32, (NDAG, BLK), 0)
              == b3_ref[0]).astype(jnp.float32)

    @pl.when(p == 0)
    def _phase0():
        aggr = dis_ref[...] * (sp_ref[0] + sp_ref[1] + g_ref[...])
        a = jnp.maximum(_dot(aggr, wa[...].T) + ba[...], 0.0)
        a2 = jnp.maximum(_dot(a, wb[...].T) + bb[...], 0.0)
        xn = _dot(a2, wc[...].T) + bc[...]
        xn_s[pl.ds(i * BLK, BLK), :] = xn

        @pl.when(i == 0)
        def _():
            yn_s[...] = jnp.zeros_like(yn_s)

        yn_s[...] += _dot_hp(onehot, xn)

    @pl.when(p == 1)
    def _phase1():
        y1 = jnp.maximum(_dot(yx_ref[...], wd1x[...].T)
                         + _dot(yn_s[...], wd1n[...].T) + bd1[...], 0.0)
        y2 = jnp.maximum(_dot(y1, wd2[...].T) + bd2[...], 0.0)
        y = _dot(y2, wd3[...].T) + bd3[...]
        z0 = jnp.sum(y, axis=0, keepdims=True)
        z1 = jnp.maximum(_dot(z0, wg1[...].T) + bg1[...], 0.0)
        z2 = jnp.maximum(_dot(z1, wg2[...].T) + bg2[...], 0.0)
        z = _dot(z2, wg3[...].T) + bg3[...]

        @pl.when(i == 0)
        def _():
            t_y = _dot(y, wpy[...].T)
            t_z = _dot(z, wpz[...].T) + bp1[...]
            lim = (lax.broadcasted_iota(jnp.int32, (NWRK, 32), 0) + 1
                   ).astype(jnp.float32)
            t_l = lim * wpl[...]
            l1 = jnp.maximum(t_y[:, None, :] + t_l[None, :, :]
                             + t_z[None], 0.0)
            l2 = jnp.maximum(
                lax.dot_general(l1, wp2[...], (((2,), (1,)), ((), ())))
                + bp2[...], 0.0)
            pr = jnp.sum(_b16(l2) * _b16(wp3[...])[None], axis=2)
            pr_ref[...] = pr + bp3[...]

        yb = lax.dot_general(onehot, y, (((0,), (0,)), ((), ())),
                             precision=jax.lax.Precision.HIGHEST)
        xn = xn_s[pl.ds(i * BLK, BLK), :]
        l1o = jnp.maximum(
            _dot(xn, wax[...].T) + _dot(yb, way[...].T)
            + _dot(z, waz[...].T) + boa[...], 0.0)
        l2o = jnp.maximum(_dot(l1o, wob[...].T) + bob[...], 0.0)
        ops_ref[...] = (jnp.sum(_b16(l2o) * _b16(woc[...]), axis=1,
                                keepdims=True) + boc[...])


def _full(shape):
    return pl.BlockSpec(shape, lambda *_: tuple(0 for _ in shape))


def kernel(x, edge_index, batch, num_ops_per_dag, op_msk, prlvl_msk, params):
    f32 = jnp.float32
    (w1a, b1a), (w1b, b1b), (w1c, b1c) = params['mlp1']
    (w2a, b2a), (w2b, b2b), (w2c, b2c) = params['mlp2']
    (wda, bda), (wdb, bdb), (wdc, bdc) = params['mlp_dag']
    (wga, bga), (wgb, bgb), (wgc, bgc) = params['mlp_global']
    (woa, boa), (wob, bob), (woc, boc) = params['mlp_op_score']
    (wpa, bpa), (wpb, bpb), (wpc, bpc) = params['mlp_prlvl_score']
    r2 = lambda b: b.reshape(1, -1)

    b3 = batch.reshape(GRID, 1, BLK)
    row1 = edge_index[0]
    col1 = edge_index[1]
    zeros1 = jnp.zeros((RPT,), f32)
    zeros8 = jnp.zeros((RPS, 8), f32)
    ones2 = jnp.ones((EPT,), f32)

    deg_parts = _sc_degree(col1, zeros1, ones2)
    dp = deg_parts.reshape(2, NPAD, 1)

    g, dis, y_x = pl.pallas_call(
        _k1_body,
        grid=(GRID,),
        in_specs=[
            pl.BlockSpec((BLK, 128), lambda i: (i, 0)),
            pl.BlockSpec((1, 1, BLK), lambda i: (i, 0, 0)),
            pl.BlockSpec((2, BLK, 1), lambda i: (0, i, 0)),
            _full((32, 128)), _full((1, 32)),
            _full((16, 32)), _full((1, 16)),
            _full((8, 16)), _full((1, 8)),
        ],
        out_specs=[
            pl.BlockSpec((BLK, 8), lambda i: (i, 0)),
            pl.BlockSpec((BLK, 1), lambda i: (i, 0)),
            _full((NDAG, 128)),
        ],
        out_shape=[
            jax.ShapeDtypeStruct((N_NODES, 8), f32),
            jax.ShapeDtypeStruct((N_NODES, 1), f32),
            jax.ShapeDtypeStruct((NDAG, 128), f32),
        ],
    )(x, b3, dp, w1a, r2(b1a), w1b, r2(b1b), w1c, r2(b1c))

    s_parts = _sc_gather_scatter(col1, row1, g, zeros8)

    ops, prlvl = pl.pallas_call(
        _k345_body,
        grid=(2, GRID),
        in_specs=[
            pl.BlockSpec((2, BLK, 8), lambda p, i: (0, i, 0)),
            pl.BlockSpec((BLK, 8), lambda p, i: (i, 0)),
            pl.BlockSpec((BLK, 1), lambda p, i: (i, 0)),
            pl.BlockSpec((1, 1, BLK), lambda p, i: (i, 0, 0)),
            _full((NDAG, 128)),
            _full((32, 8)), _full((1, 32)),
            _full((16, 32)), _full((1, 16)),
            _full((16, 16)), _full((1, 16)),
            _full((32, 128)), _full((32, 16)), _full((1, 32)),
            _full((16, 32)), _full((1, 16)),
            _full((16, 16)), _full((1, 16)),
            _full((32, 16)), _full((1, 32)),
            _full((16, 32)), _full((1, 16)),
            _full((16, 16)), _full((1, 16)),
            _full((1, 32)), _full((32, 16)), _full((32, 16)), _full((1, 32)),
            _full((16, 32)), _full((1, 16)),
            _full((1, 16)), _full((1, 1)),
            _full((32, 16)), _full((32, 16)), _full((32, 16)), _full((1, 32)),
            _full((16, 32)), _full((1, 16)),
            _full((1, 16)), _full((1, 1)),
        ],
        out_specs=[
            pl.BlockSpec((BLK, 1), lambda p, i: (p * i + (1 - p) * GRID, 0)),
            _full((NDAG, NWRK)),
        ],
        out_shape=[
            jax.ShapeDtypeStruct((N_NODES + BLK, 1), f32),
            jax.ShapeDtypeStruct((NDAG, NWRK), f32),
        ],
        scratch_shapes=[
            pltpu.VMEM((N_NODES, 16), f32),
            pltpu.VMEM((NDAG, 16), f32),
        ],
    )(s_parts, g, dis, b3, y_x,
      w2a, r2(b2a), w2b, r2(b2b), w2c, r2(b2c),
      wda[:, :128], wda[:, 128:], r2(bda),
      wdb, r2(bdb), wdc, r2(bdc),
      wga, r2(bga), wgb, r2(bgb), wgc, r2(bgc),
      wpa[:, 0:1].T, wpa[:, 1:17], wpa[:, 17:33], r2(bpa),
      wpb, r2(bpb), wpc, bpc.reshape(1, 1),
      woa[:, 0:16], woa[:, 16:32], woa[:, 32:48], r2(boa),
      wob, r2(bob), woc, boc.reshape(1, 1))

    return ops[:N_NODES, 0], prlvl


# speedup vs baseline: 1.4372x; 1.1772x over previous
import functools

import jax
import jax.numpy as jnp
from jax import lax
from jax.experimental import pallas as pl
from jax.experimental.pallas import tpu as pltpu
from jax.experimental.pallas import tpu_sc as plsc

N_NODES = 10000
NPAD = 10240
NDAG = 100
NWRK = 50
CH = 80
NCH = 125
NTILES = 32
EPT = NCH * CH
BLK = 2000
GRID = N_NODES // BLK
RPT = NPAD // 16
RPS = N_NODES // 16


@functools.cache
def _sc_kernels():
    mesh = plsc.VectorSubcoreMesh(core_axis_name="c", subcore_axis_name="s")

    @functools.partial(
        pl.kernel,
        out_type=jax.ShapeDtypeStruct((2, NPAD), jnp.float32),
        mesh=mesh,
        scratch_types=[
            pltpu.VMEM((EPT,), jnp.int32),
            pltpu.VMEM((EPT,), jnp.float32),
            pltpu.VMEM_SHARED((NPAD,), jnp.float32),
            pltpu.SemaphoreType.DMA,
        ],
    )
    def sc_degree(col_hbm, zeros_hbm, ones_hbm, out_hbm, idx_v, ones_v,
                  deg_sh, sem):
        c = lax.axis_index("c")
        s = lax.axis_index("s")
        wid = c * 16 + s
        pltpu.async_copy(col_hbm.at[pl.ds(wid * EPT, EPT)], idx_v, sem)
        pltpu.async_copy(ones_hbm, ones_v, sem)
        pltpu.sync_copy(zeros_hbm, deg_sh.at[pl.ds(s * RPT, RPT)])
        pltpu.make_async_copy(col_hbm.at[pl.ds(wid * EPT, EPT)], idx_v,
                              sem).wait()
        pltpu.make_async_copy(ones_hbm, ones_v, sem).wait()
        plsc.subcore_barrier()
        pltpu.sync_copy(ones_v, deg_sh.at[idx_v], add=True)
        plsc.subcore_barrier()
        pltpu.sync_copy(deg_sh.at[pl.ds(s * RPT, RPT)],
                        out_hbm.at[c, pl.ds(s * RPT, RPT)])

    @functools.partial(
        pl.kernel,
        out_type=jax.ShapeDtypeStruct((2, N_NODES, 8), jnp.float32),
        mesh=mesh,
        scratch_types=[
            pltpu.VMEM((EPT,), jnp.int32),
            pltpu.VMEM((EPT,), jnp.int32),
            pltpu.VMEM((EPT, 8), jnp.float32),
            pltpu.VMEM_SHARED((N_NODES, 8), jnp.float32),
            pltpu.SemaphoreType.DMA,
        ],
        compiler_params=pltpu.CompilerParams(use_tc_tiling_on_sc=False),
    )
    def sc_gather_scatter(col_hbm, row_hbm, g_hbm, zeros_hbm, out_hbm,
                          col_v, row_v, rows_v, acc_sh, sem):
        c = lax.axis_index("c")
        s = lax.axis_index("s")
        wid = c * 16 + s
        pltpu.async_copy(col_hbm.at[pl.ds(wid * EPT, EPT)], col_v, sem)
        pltpu.async_copy(row_hbm.at[pl.ds(wid * EPT, EPT)], row_v, sem)
        pltpu.sync_copy(zeros_hbm, acc_sh.at[pl.ds(s * RPS, RPS)])
        pltpu.make_async_copy(col_hbm.at[pl.ds(wid * EPT, EPT)], col_v,
                              sem).wait()
        pltpu.make_async_copy(row_hbm.at[pl.ds(wid * EPT, EPT)], row_v,
                              sem).wait()
        pltpu.async_copy(g_hbm.at[col_v], rows_v, sem).wait()
        plsc.subcore_barrier()
        pltpu.sync_copy(rows_v, acc_sh.at[row_v], add=True)
        plsc.subcore_barrier()
        pltpu.sync_copy(acc_sh.at[pl.ds(s * RPS, RPS)],
                        out_hbm.at[c, pl.ds(s * RPS, RPS)])

    return sc_degree, sc_gather_scatter


def _sc_degree(col1, zeros1, ones2):
    return _sc_kernels()[0](col1, zeros1, ones2)


def _sc_gather_scatter(col1, row1, g, zeros8):
    return _sc_kernels()[1](col1, row1, g, zeros8)


def _dot(a, b):
    return jax.lax.dot_general(a, b, (((a.ndim - 1,), (0,)), ((), ())))


def _dot_hp(a, b):
    return jax.lax.dot_general(a, b, (((a.ndim - 1,), (0,)), ((), ())),
                               precision=jax.lax.Precision.HIGHEST)


def _b16(v):
    return v.astype(jnp.bfloat16).astype(jnp.float32)


def _dot_oh(onehot, x):
    dims = (((1,), (0,)), ((), ()))
    oh = onehot.astype(jnp.bfloat16)
    x1 = x.astype(jnp.bfloat16)
    r1 = x - x1.astype(jnp.float32)
    x2 = r1.astype(jnp.bfloat16)
    x3 = (r1 - x2.astype(jnp.float32)).astype(jnp.bfloat16)
    f32 = jnp.float32
    return ((jax.lax.dot_general(oh, x3, dims, preferred_element_type=f32)
             + jax.lax.dot_general(oh, x2, dims, preferred_element_type=f32))
            + jax.lax.dot_general(oh, x1, dims, preferred_element_type=f32))


def _k1_body(x_ref, b3_ref, dp_ref, wa, ba, wb, bb, wc, bc,
             g_ref, dis_ref, yx_ref):
    i = pl.program_id(0)
    xb = x_ref[...]
    a = jnp.maximum(_dot(xb, wa[...].T) + ba[...], 0.0)
    a = jnp.maximum(_dot(a, wb[...].T) + bb[...], 0.0)
    h = _dot(a, wc[...].T) + bc[...]
    deg = dp_ref[0] + dp_ref[1] + 1.0
    dis = 1.0 / jnp.sqrt(deg)
    dis_ref[...] = dis
    g_ref[...] = dis * h
    onehot = (lax.broadcasted_iota(jnp.int32, (NDAG, BLK), 0)
              == b3_ref[0]).astype(jnp.float32)

    @pl.when(i == 0)
    def _():
        yx_ref[...] = jnp.zeros_like(yx_ref)

    yx_ref[...] += _dot_oh(onehot, xb)


def _k345_body(sp_ref, g_ref, dis_ref, b3_ref, yx_ref,
               wa, ba, wb, bb, wc, bc,
               wd1x, wd1n, bd1, wd2, bd2, wd3, bd3,
               wg1, bg1, wg2, bg2, wg3, bg3,
               wpl, wpy, wpz, bp1, wp2, bp2, wp3, bp3,
               wax, way, waz, boa, wob, bob, woc, boc,
               ops_ref, pr_ref, xn_s, yn_s):
    p = pl.program_id(0)
    i = pl.program_id(1)
    onehot = (lax.broadcasted_iota(jnp.int32, (NDAG, BLK), 0)
              == b3_ref[0]).astype(jnp.float32)

    @pl.when(p == 0)
    def _phase0():
        aggr = dis_ref[...] * (sp_ref[0] + sp_ref[1] + g_ref[...])
        a = jnp.maximum(_dot(aggr, wa[...].T) + ba[...], 0.0)
        a2 = jnp.maximum(_dot(a, wb[...].T) + bb[...], 0.0)
        xn = _dot(a2, wc[...].T) + bc[...]
        xn_s[pl.ds(i * BLK, BLK), :] = xn

        @pl.when(i == 0)
        def _():
            yn_s[...] = jnp.zeros_like(yn_s)

        yn_s[...] += _dot_hp(onehot, xn)

    @pl.when(p == 1)
    def _phase1():
        y1 = jnp.maximum(_dot(yx_ref[...], wd1x[...].T)
                         + _dot(yn_s[...], wd1n[...].T) + bd1[...], 0.0)
        y2 = jnp.maximum(_dot(y1, wd2[...].T) + bd2[...], 0.0)
        y = _dot(y2, wd3[...].T) + bd3[...]
        z0 = jnp.sum(y, axis=0, keepdims=True)
        z1 = jnp.maximum(_dot(z0, wg1[...].T) + bg1[...], 0.0)
        z2 = jnp.maximum(_dot(z1, wg2[...].T) + bg2[...], 0.0)
        z = _dot(z2, wg3[...].T) + bg3[...]

        @pl.when(i == 0)
        def _():
            t_y = _dot(y, wpy[...].T)
            t_z = _dot(z, wpz[...].T) + bp1[...]
            lim = (lax.broadcasted_iota(jnp.int32, (NWRK, 32), 0) + 1
                   ).astype(jnp.float32)
            t_l = lim * wpl[...]
            l1 = jnp.maximum(t_y[:, None, :] + t_l[None, :, :]
                             + t_z[None], 0.0)
            l2 = jnp.maximum(
                lax.dot_general(l1, wp2[...], (((2,), (1,)), ((), ())))
                + bp2[...], 0.0)
            pr = jnp.sum(_b16(l2) * _b16(wp3[...])[None], axis=2)
            pr_ref[...] = pr + bp3[...]

        yb = lax.dot_general(onehot, y, (((0,), (0,)), ((), ())),
                             precision=jax.lax.Precision.HIGHEST)
        xn = xn_s[pl.ds(i * BLK, BLK), :]
        l1o = jnp.maximum(
            _dot(xn, wax[...].T) + _dot(yb, way[...].T)
            + _dot(z, waz[...].T) + boa[...], 0.0)
        l2o = jnp.maximum(_dot(l1o, wob[...].T) + bob[...], 0.0)
        ops_ref[...] = (jnp.sum(_b16(l2o) * _b16(woc[...]), axis=1,
                                keepdims=True) + boc[...])


def _full(shape):
    return pl.BlockSpec(shape, lambda *_: tuple(0 for _ in shape))


def kernel(x, edge_index, batch, num_ops_per_dag, op_msk, prlvl_msk, params):
    f32 = jnp.float32
    (w1a, b1a), (w1b, b1b), (w1c, b1c) = params['mlp1']
    (w2a, b2a), (w2b, b2b), (w2c, b2c) = params['mlp2']
    (wda, bda), (wdb, bdb), (wdc, bdc) = params['mlp_dag']
    (wga, bga), (wgb, bgb), (wgc, bgc) = params['mlp_global']
    (woa, boa), (wob, bob), (woc, boc) = params['mlp_op_score']
    (wpa, bpa), (wpb, bpb), (wpc, bpc) = params['mlp_prlvl_score']
    r2 = lambda b: b.reshape(1, -1)

    b3 = batch.reshape(GRID, 1, BLK)
    row1 = edge_index[0]
    col1 = edge_index[1]
    zeros1 = jnp.zeros((RPT,), f32)
    zeros8 = jnp.zeros((RPS, 8), f32)
    ones2 = jnp.ones((EPT,), f32)

    deg_parts = _sc_degree(col1, zeros1, ones2)
    dp = deg_parts.reshape(2, NPAD, 1)

    g, dis, y_x = pl.pallas_call(
        _k1_body,
        grid=(GRID,),
        in_specs=[
            pl.BlockSpec((BLK, 128), lambda i: (i, 0)),
            pl.BlockSpec((1, 1, BLK), lambda i: (i, 0, 0)),
            pl.BlockSpec((2, BLK, 1), lambda i: (0, i, 0)),
            _full((32, 128)), _full((1, 32)),
            _full((16, 32)), _full((1, 16)),
            _full((8, 16)), _full((1, 8)),
        ],
        out_specs=[
            pl.BlockSpec((BLK, 8), lambda i: (i, 0)),
            pl.BlockSpec((BLK, 1), lambda i: (i, 0)),
            _full((NDAG, 128)),
        ],
        out_shape=[
            jax.ShapeDtypeStruct((N_NODES, 8), f32),
            jax.ShapeDtypeStruct((N_NODES, 1), f32),
            jax.ShapeDtypeStruct((NDAG, 128), f32),
        ],
    )(x, b3, dp, w1a, r2(b1a), w1b, r2(b1b), w1c, r2(b1c))

    s_parts = _sc_gather_scatter(col1, row1, g, zeros8)

    ops, prlvl = pl.pallas_call(
        _k345_body,
        grid=(2, GRID),
        in_specs=[
            pl.BlockSpec((2, BLK, 8), lambda p, i: (0, i, 0)),
            pl.BlockSpec((BLK, 8), lambda p, i: (i, 0)),
            pl.BlockSpec((BLK, 1), lambda p, i: (i, 0)),
            pl.BlockSpec((1, 1, BLK), lambda p, i: (i, 0, 0)),
            _full((NDAG, 128)),
            _full((32, 8)), _full((1, 32)),
            _full((16, 32)), _full((1, 16)),
            _full((16, 16)), _full((1, 16)),
            _full((32, 128)), _full((32, 16)), _full((1, 32)),
            _full((16, 32)), _full((1, 16)),
            _full((16, 16)), _full((1, 16)),
            _full((32, 16)), _full((1, 32)),
            _full((16, 32)), _full((1, 16)),
            _full((16, 16)), _full((1, 16)),
            _full((1, 32)), _full((32, 16)), _full((32, 16)), _full((1, 32)),
            _full((16, 32)), _full((1, 16)),
            _full((1, 16)), _full((1, 1)),
            _full((32, 16)), _full((32, 16)), _full((32, 16)), _full((1, 32)),
            _full((16, 32)), _full((1, 16)),
            _full((1, 16)), _full((1, 1)),
        ],
        out_specs=[
            pl.BlockSpec((BLK, 1), lambda p, i: (p * i + (1 - p) * GRID, 0)),
            _full((NDAG, NWRK)),
        ],
        out_shape=[
            jax.ShapeDtypeStruct((N_NODES + BLK, 1), f32),
            jax.ShapeDtypeStruct((NDAG, NWRK), f32),
        ],
        scratch_shapes=[
            pltpu.VMEM((N_NODES, 16), f32),
            pltpu.VMEM((NDAG, 16), f32),
        ],
    )(s_parts, g, dis, b3, y_x,
      w2a, r2(b2a), w2b, r2(b2b), w2c, r2(b2c),
      wda[:, :128], wda[:, 128:], r2(bda),
      wdb, r2(bdb), wdc, r2(bdc),
      wga, r2(bga), wgb, r2(bgb), wgc, r2(bgc),
      wpa[:, 0:1].T, wpa[:, 1:17], wpa[:, 17:33], r2(bpa),
      wpb, r2(bpb), wpc, bpc.reshape(1, 1),
      woa[:, 0:16], woa[:, 16:32], woa[:, 32:48], r2(boa),
      wob, r2(bob), woc, boc.reshape(1, 1))

    return ops[:N_NODES, 0], prlvl
